# baseline (speedup 1.0000x reference)
import functools

import jax
import jax.numpy as jnp
from jax import lax
from jax.experimental import pallas as pl
from jax.experimental.pallas import tpu as pltpu
from jax.experimental.pallas import tpu_sc as plsc

_B = 16384
_D = 64
_CHUNK = 128


def kernel(user_table, item_table, user_idx, item_idx):
    info = plsc.get_sparse_core_info()
    nw = info.num_cores * info.num_subcores
    bpw = _B // nw
    nchunk = bpw // _CHUNK

    mesh = plsc.VectorSubcoreMesh(core_axis_name="c", subcore_axis_name="s")

    @functools.partial(
        pl.kernel,
        mesh=mesh,
        out_type=jax.ShapeDtypeStruct((2 * _B, _D), jnp.float32),
        scratch_types=[
            pltpu.VMEM((nchunk, _CHUNK), jnp.int32),
            pltpu.VMEM((nchunk, _CHUNK), jnp.int32),
            pltpu.VMEM((bpw, _D), jnp.float32),
            pltpu.VMEM((bpw, _D), jnp.float32),
            pltpu.SemaphoreType.DMA,
        ],
        compiler_params=pltpu.CompilerParams(use_tc_tiling_on_sc=False),
    )
    def _emb(ut, it, ui, ii, out, uidx_v, iidx_v, urows_v, irows_v, sem):
        wid = lax.axis_index("s") * info.num_cores + lax.axis_index("c")
        base = wid * bpw
        pltpu.sync_copy(ui.at[wid], uidx_v)
        pltpu.sync_copy(ii.at[wid], iidx_v)
        copies = []
        for j in range(nchunk):
            copies.append(pltpu.async_copy(
                ut.at[uidx_v.at[j]], urows_v.at[pl.ds(j * _CHUNK, _CHUNK)], sem))
            copies.append(pltpu.async_copy(
                it.at[iidx_v.at[j]], irows_v.at[pl.ds(j * _CHUNK, _CHUNK)], sem))
        for c in copies:
            c.wait()
        pltpu.sync_copy(urows_v, out.at[pl.ds(base, bpw)])
        pltpu.sync_copy(irows_v, out.at[pl.ds(_B + base, bpw)])

    ui3 = user_idx.reshape(nw, nchunk, _CHUNK).astype(jnp.int32)
    ii3 = item_idx.reshape(nw, nchunk, _CHUNK).astype(jnp.int32)
    out = _emb(user_table, item_table, ui3, ii3)
    return out.reshape(2, _B, _D)
